# baseline (speedup 1.0000x reference)
import functools

import jax
import jax.numpy as jnp
from jax import lax
from jax.experimental import pallas as pl
from jax.experimental.pallas import tpu as pltpu
from jax.experimental.pallas import tpu_sc as plsc

D_MODEL = 1024
D_FF = 2048
NUM_E = 8
ROW_TILE = 256
D_TILE = 512

NC = 2
NS = 16
NW = NC * NS

_INTERPRET = False


def _silu(v):
    return v * (1.0 / (1.0 + jnp.exp(-v)))


def _router_body(x_ref, gate_ref, a0_ref, a1_ref, w0_ref, w1c_ref,
                 h0_ref, h1_ref):
    x = x_ref[...]
    rows = x.shape[0]
    logits = jnp.dot(x, gate_ref[...], preferred_element_type=jnp.float32)
    iota = lax.broadcasted_iota(jnp.int32, (rows, NUM_E), 1)
    m0 = jnp.max(logits, axis=1, keepdims=True)
    a0 = jnp.min(jnp.where(logits == m0, iota, NUM_E), axis=1, keepdims=True)
    masked = jnp.where(iota == a0, jnp.float32(-1e30), logits)
    m1 = jnp.max(masked, axis=1, keepdims=True)
    a1 = jnp.min(jnp.where(masked == m1, iota, NUM_E), axis=1, keepdims=True)
    t = jnp.exp(m1 - m0)
    a0_ref[...] = a0
    a1_ref[...] = a1
    w0_ref[...] = 1.0 / (1.0 + t)
    w1c_ref[...] = t / (1.0 + t)
    iota16 = lax.broadcasted_iota(jnp.int32, (rows, 16), 1)
    h0_ref[...] = jnp.sum(jnp.where(iota16 == a0, 1, 0),
                          axis=0, keepdims=True)[None]
    h1_ref[...] = jnp.sum(jnp.where(iota16 == a1, 1, 0),
                          axis=0, keepdims=True)[None]


def _router(x_flat, gate_w):
    n = x_flat.shape[0]
    n_tiles = n // ROW_TILE
    return pl.pallas_call(
        _router_body,
        grid=(n_tiles,),
        in_specs=[
            pl.BlockSpec((ROW_TILE, D_MODEL), lambda t: (t, 0)),
            pl.BlockSpec((D_MODEL, NUM_E), lambda t: (0, 0)),
        ],
        out_specs=[
            pl.BlockSpec((ROW_TILE, 1), lambda t: (t, 0)),
            pl.BlockSpec((ROW_TILE, 1), lambda t: (t, 0)),
            pl.BlockSpec((ROW_TILE, 1), lambda t: (t, 0)),
            pl.BlockSpec((ROW_TILE, 1), lambda t: (t, 0)),
            pl.BlockSpec((1, 1, 16), lambda t: (t, 0, 0)),
            pl.BlockSpec((1, 1, 16), lambda t: (t, 0, 0)),
        ],
        out_shape=[
            jax.ShapeDtypeStruct((n, 1), jnp.int32),
            jax.ShapeDtypeStruct((n, 1), jnp.int32),
            jax.ShapeDtypeStruct((n, 1), jnp.float32),
            jax.ShapeDtypeStruct((n, 1), jnp.float32),
            jax.ShapeDtypeStruct((n_tiles, 1, 16), jnp.int32),
            jax.ShapeDtypeStruct((n_tiles, 1, 16), jnp.int32),
        ],
        compiler_params=pltpu.CompilerParams(
            dimension_semantics=("arbitrary",),
        ),
        interpret=_INTERPRET,
    )(x_flat, gate_w)


def _shared_body(x_ref, w1_ref, w2_ref, w3_ref, sh_ref):
    x = x_ref[...]
    gate = _silu(jnp.dot(x, w1_ref[...], preferred_element_type=jnp.float32))
    up = jnp.dot(x, w2_ref[...], preferred_element_type=jnp.float32)
    sh_ref[...] = jnp.dot(gate * up, w3_ref[...], preferred_element_type=jnp.float32)


def _shared(x_part, sw1, sw2, sw3):
    n = x_part.shape[0]
    n_tiles = n // ROW_TILE
    return pl.pallas_call(
        _shared_body,
        grid=(n_tiles,),
        in_specs=[
            pl.BlockSpec((ROW_TILE, D_MODEL), lambda t: (t, 0)),
            pl.BlockSpec((D_MODEL, D_FF), lambda t: (0, 0)),
            pl.BlockSpec((D_MODEL, D_FF), lambda t: (0, 0)),
            pl.BlockSpec((D_FF, D_MODEL), lambda t: (0, 0)),
        ],
        out_specs=pl.BlockSpec((ROW_TILE, D_MODEL), lambda t: (t, 0)),
        out_shape=jax.ShapeDtypeStruct((n, D_MODEL), jnp.float32),
        compiler_params=pltpu.CompilerParams(
            dimension_semantics=("arbitrary",),
        ),
        interpret=_INTERPRET,
    )(x_part, sw1, sw2, sw3)


def _wid():
    return lax.axis_index("s") * NC + lax.axis_index("c")


def _splat(v, k):
    return v[jnp.full((16,), k, jnp.int32)]


def _cumsum16(v, iota):
    for k in (1, 2, 4, 8):
        sv = v[jnp.maximum(iota - k, 0)]
        v = v + jnp.where(iota >= k, sv, 0)
    return v


def _make_sort_dispatch_kernel(n_tok, n_assign, te_len, pad_total):
    chunk = n_assign // NW
    rows = 32
    n_rows_chunks = chunk // rows
    mesh = plsc.VectorSubcoreMesh(core_axis_name="c", subcore_axis_name="s")

    @functools.partial(
        pl.kernel,
        out_type=[
            jax.ShapeDtypeStruct((n_assign // rows, rows), jnp.int32),
            jax.ShapeDtypeStruct((te_len,), jnp.int32),
            jax.ShapeDtypeStruct((te_len,), jnp.int32),
            jax.ShapeDtypeStruct((pad_total, D_MODEL), jnp.float32),
        ],
        mesh=mesh,
        scratch_types=[
            pltpu.VMEM((chunk,), jnp.int32),
            pltpu.VMEM((NW, 16), jnp.int32),
            pltpu.VMEM((n_rows_chunks, rows), jnp.int32),
            pltpu.VMEM((te_len,), jnp.int32),
            pltpu.VMEM((te_len,), jnp.int32),
            pltpu.VMEM((rows, D_MODEL), jnp.float32),
            pltpu.VMEM((rows, D_MODEL), jnp.float32),
            pltpu.SemaphoreType.DMA,
            pltpu.SemaphoreType.DMA,
        ],
    )
    def sort_dispatch_kernel(e0_hbm, e1_hbm, h0_hbm, h1_hbm, x_hbm, dst_hbm,
                             te_hbm, af_hbm, disp_hbm, ids_v, allh_v, dst_v,
                             te_v, af_v, rows_v0, rows_v1, sem0, sem1):
        w = _wid()

        @pl.when(w < NS)
        def _():
            pltpu.sync_copy(e0_hbm.at[pl.ds((w % NS) * chunk, chunk)], ids_v)

        @pl.when(w >= NS)
        def _():
            pltpu.sync_copy(e1_hbm.at[pl.ds((w % NS) * chunk, chunk)], ids_v)

        pltpu.sync_copy(h0_hbm, allh_v.at[pl.ds(0, NS)])
        pltpu.sync_copy(h1_hbm, allh_v.at[pl.ds(NS, NS)])
        iota = lax.iota(jnp.int32, 16)
        tot = jnp.zeros((16,), jnp.int32)
        pre = jnp.zeros((16,), jnp.int32)
        for ww in range(NW):
            row = allh_v[ww]
            tot = tot + row
            pre = pre + row * jnp.where(ww < w, 1, 0)
        padded = (tot + (D_TILE - 1)) & jnp.int32(-D_TILE)
        csum = _cumsum16(padded, iota)
        off = csum - padded
        start = off + pre

        cnt = jnp.zeros((16,), jnp.int32)
        for j in range(chunk // 16):
            v = ids_v[pl.ds(j * 16, 16)]
            rank = jnp.zeros((16,), jnp.int32)
            for k in range(1, 16):
                sv = v[jnp.maximum(iota - k, 0)]
                rank = rank + jnp.where((iota >= k) & (sv == v), 1, 0)
            dst_v[j // 2, pl.ds((j % 2) * 16, 16)] = start[v] + cnt[v] + rank
            cc = jnp.zeros((16,), jnp.int32)
            for k in range(16):
                cc = cc + jnp.where(iota == _splat(v, k), 1, 0)
            cnt = cnt + cc
        pltpu.sync_copy(dst_v, dst_hbm.at[pl.ds(w * n_rows_chunks, n_rows_chunks)])

        tok_per_slotblock = n_tok // NS
        bufs = (rows_v0, rows_v1)
        sems = (sem0, sem1)
        cps = [None, None]
        for r in range(n_rows_chunks):
            b = r % 2
            if cps[b] is not None:
                cps[b].wait()
            tb = (w % NS) * tok_per_slotblock + r * rows
            pltpu.sync_copy(x_hbm.at[pl.ds(tb, rows)], bufs[b])
            cps[b] = pltpu.async_copy(bufs[b], disp_hbm.at[dst_v.at[r]], sems[b])
        for cp in cps:
            if cp is not None:
                cp.wait()

        @pl.when(w == 0)
        def _():
            used_end = off + tot
            for g in range(te_len // 16):
                tstart = (iota + g * 16) * D_TILE
                acc = jnp.zeros((16,), jnp.int32)
                for e in range(NUM_E):
                    acc = acc + jnp.where(_splat(off, e) <= tstart, 1, 0)
                ex = acc - 1
                te_v[pl.ds(g * 16, 16)] = ex
                af_v[pl.ds(g * 16, 16)] = jnp.where(tstart < used_end[ex], 1, 0)
            pltpu.sync_copy(te_v, te_hbm)
            pltpu.sync_copy(af_v, af_hbm)

    return sort_dispatch_kernel


def _expert_ffn_body(te_ref, af_ref, x_ref, w1_ref, w2_ref, w3_ref, out_ref):
    del te_ref
    t = pl.program_id(0)

    @pl.when(af_ref[t] == 1)
    def _():
        x = x_ref[...]
        gate = _silu(jnp.dot(x, w1_ref[0], preferred_element_type=jnp.float32))
        up = jnp.dot(x, w2_ref[0], preferred_element_type=jnp.float32)
        out_ref[...] = jnp.dot(gate * up, w3_ref[0], preferred_element_type=jnp.float32)


def _expert_ffn(te, af, disp, ew1, ew2, ew3, n_tiles):
    pad_total = disp.shape[0]
    grid_spec = pltpu.PrefetchScalarGridSpec(
        num_scalar_prefetch=2,
        grid=(n_tiles,),
        in_specs=[
            pl.BlockSpec((D_TILE, D_MODEL), lambda t, te, af: (t, 0)),
            pl.BlockSpec((1, D_MODEL, D_FF), lambda t, te, af: (te[t], 0, 0)),
            pl.BlockSpec((1, D_MODEL, D_FF), lambda t, te, af: (te[t], 0, 0)),
            pl.BlockSpec((1, D_FF, D_MODEL), lambda t, te, af: (te[t], 0, 0)),
        ],
        out_specs=pl.BlockSpec((D_TILE, D_MODEL), lambda t, te, af: (t, 0)),
    )
    return pl.pallas_call(
        _expert_ffn_body,
        grid_spec=grid_spec,
        out_shape=jax.ShapeDtypeStruct((pad_total, D_MODEL), jnp.float32),
        compiler_params=pltpu.CompilerParams(
            dimension_semantics=("arbitrary",),
            vmem_limit_bytes=100 * 1024 * 1024,
        ),
        interpret=_INTERPRET,
    )(te, af, disp, ew1, ew2, ew3)


def _make_gather_kernel(n_assign, pad_total):
    rows = 32
    n_rows_chunks = n_assign // rows // NW
    mesh = plsc.VectorSubcoreMesh(core_axis_name="c", subcore_axis_name="s")

    @functools.partial(
        pl.kernel,
        out_type=jax.ShapeDtypeStruct((n_assign, D_MODEL), jnp.float32),
        mesh=mesh,
        scratch_types=[
            pltpu.VMEM((rows, D_MODEL), jnp.float32),
            pltpu.VMEM((rows, D_MODEL), jnp.float32),
            pltpu.VMEM((n_rows_chunks, rows), jnp.int32),
            pltpu.SemaphoreType.DMA,
            pltpu.SemaphoreType.DMA,
        ],
    )
    def gather_kernel(eout_hbm, dst_hbm, g_hbm, rows_v0, rows_v1, idx_v,
                      sem0, sem1):
        w = _wid()
        pltpu.sync_copy(
            dst_hbm.at[pl.ds(w * n_rows_chunks, n_rows_chunks)], idx_v)
        bufs = (rows_v0, rows_v1)
        sems = (sem0, sem1)
        cps = [None, None]
        for c in range(n_rows_chunks):
            b = c % 2
            cps[b] = pltpu.async_copy(eout_hbm.at[idx_v.at[c]], bufs[b], sems[b])
            if c > 0:
                cps[1 - b].wait()
                rr = w * n_rows_chunks + c - 1
                pltpu.sync_copy(bufs[1 - b], g_hbm.at[pl.ds(rr * rows, rows)])
        last = n_rows_chunks - 1
        cps[last % 2].wait()
        pltpu.sync_copy(bufs[last % 2],
                        g_hbm.at[pl.ds((w * n_rows_chunks + last) * rows, rows)])

    return gather_kernel


def _combine_body(sl_ref, sh_ref, g0_ref, g1_ref, w0_ref, w1_ref, out_ref):
    t = pl.program_id(0)
    nh = pl.num_programs(0) // 2
    routed = w0_ref[...] * g0_ref[...] + w1_ref[...] * g1_ref[...]

    @pl.when(t < nh)
    def _():
        out_ref[...] = sl_ref[...] + routed

    @pl.when(t >= nh)
    def _():
        out_ref[...] = sh_ref[...] + routed


def _combine(shared_lo, shared_hi, g, w0c, w1c, n_tok):
    n_tiles = n_tok // ROW_TILE
    nh = n_tiles // 2
    return pl.pallas_call(
        _combine_body,
        grid=(n_tiles,),
        in_specs=[
            pl.BlockSpec((ROW_TILE, D_MODEL), lambda t: (t % nh, 0)),
            pl.BlockSpec((ROW_TILE, D_MODEL), lambda t: (t % nh, 0)),
            pl.BlockSpec((ROW_TILE, D_MODEL), lambda t: (t, 0)),
            pl.BlockSpec((ROW_TILE, D_MODEL), lambda t: (t + n_tok // ROW_TILE, 0)),
            pl.BlockSpec((ROW_TILE, 1), lambda t: (t, 0)),
            pl.BlockSpec((ROW_TILE, 1), lambda t: (t, 0)),
        ],
        out_specs=pl.BlockSpec((ROW_TILE, D_MODEL), lambda t: (t, 0)),
        out_shape=jax.ShapeDtypeStruct((n_tok, D_MODEL), jnp.float32),
        compiler_params=pltpu.CompilerParams(
            dimension_semantics=("arbitrary",),
        ),
        interpret=_INTERPRET,
    )(shared_lo, shared_hi, g, g, w0c, w1c)


def kernel(x, shared_w1, shared_w2, shared_w3, expert_w1, expert_w2, expert_w3, gate_w):
    Bn, Tn, C = x.shape
    n_tok = Bn * Tn
    n_assign = 2 * n_tok
    pad_total = n_assign + NUM_E * D_TILE
    n_tiles = n_assign // D_TILE + NUM_E
    te_len = 64

    x_flat = x.reshape(n_tok, C)

    a0, a1, w0c, w1c, h0, h1 = _router(x_flat, gate_w)

    dst2d, te, af, disp = _make_sort_dispatch_kernel(
        n_tok, n_assign, te_len, pad_total)(
            a0.reshape(-1), a1.reshape(-1),
            h0.reshape(NS, 16), h1.reshape(NS, 16), x_flat)
    half = n_tok // 2
    shared_lo = _shared(x_flat[:half], shared_w1, shared_w2, shared_w3)
    shared_hi = _shared(x_flat[half:], shared_w1, shared_w2, shared_w3)
    dep = (shared_lo[0, 0] * 0.0).astype(jnp.int32)
    eout = _expert_ffn(te, af + dep, disp, expert_w1, expert_w2, expert_w3, n_tiles)
    g = _make_gather_kernel(n_assign, pad_total)(eout, dst2d)
    out = _combine(shared_lo, shared_hi, g, w0c, w1c, n_tok)

    final_out = out.reshape(Bn, Tn, C)
    aux_loss = jnp.array(0.0, dtype=jnp.float32)
    return (final_out, aux_loss)


# speedup vs baseline: 1.1136x; 1.1136x over previous
import functools

import jax
import jax.numpy as jnp
from jax import lax
from jax.experimental import pallas as pl
from jax.experimental.pallas import tpu as pltpu
from jax.experimental.pallas import tpu_sc as plsc

D_MODEL = 1024
D_FF = 2048
NUM_E = 8
ROW_TILE = 256
D_TILE = 512

NC = 2
NS = 16
NW = NC * NS

_INTERPRET = False


def _silu(v):
    return v * (1.0 / (1.0 + jnp.exp(-v)))


def _router_body(x_ref, gate_ref, a0_ref, a1_ref, w0_ref, w1c_ref,
                 h0_ref, h1_ref):
    x = x_ref[...]
    rows = x.shape[0]
    logits = jnp.dot(x, gate_ref[...], preferred_element_type=jnp.float32)
    iota = lax.broadcasted_iota(jnp.int32, (rows, NUM_E), 1)
    m0 = jnp.max(logits, axis=1, keepdims=True)
    a0 = jnp.min(jnp.where(logits == m0, iota, NUM_E), axis=1, keepdims=True)
    masked = jnp.where(iota == a0, jnp.float32(-1e30), logits)
    m1 = jnp.max(masked, axis=1, keepdims=True)
    a1 = jnp.min(jnp.where(masked == m1, iota, NUM_E), axis=1, keepdims=True)
    t = jnp.exp(m1 - m0)
    a0_ref[...] = a0
    a1_ref[...] = a1
    w0_ref[...] = 1.0 / (1.0 + t)
    w1c_ref[...] = t / (1.0 + t)
    iota16 = lax.broadcasted_iota(jnp.int32, (rows, 16), 1)
    h0_ref[...] = jnp.sum(jnp.where(iota16 == a0, 1, 0),
                          axis=0, keepdims=True)[None]
    h1_ref[...] = jnp.sum(jnp.where(iota16 == a1, 1, 0),
                          axis=0, keepdims=True)[None]


def _router(x_flat, gate_w):
    n = x_flat.shape[0]
    n_tiles = n // ROW_TILE
    return pl.pallas_call(
        _router_body,
        grid=(n_tiles,),
        in_specs=[
            pl.BlockSpec((ROW_TILE, D_MODEL), lambda t: (t, 0)),
            pl.BlockSpec((D_MODEL, NUM_E), lambda t: (0, 0)),
        ],
        out_specs=[
            pl.BlockSpec((ROW_TILE, 1), lambda t: (t, 0)),
            pl.BlockSpec((ROW_TILE, 1), lambda t: (t, 0)),
            pl.BlockSpec((ROW_TILE, 1), lambda t: (t, 0)),
            pl.BlockSpec((ROW_TILE, 1), lambda t: (t, 0)),
            pl.BlockSpec((1, 1, 16), lambda t: (t, 0, 0)),
            pl.BlockSpec((1, 1, 16), lambda t: (t, 0, 0)),
        ],
        out_shape=[
            jax.ShapeDtypeStruct((n, 1), jnp.int32),
            jax.ShapeDtypeStruct((n, 1), jnp.int32),
            jax.ShapeDtypeStruct((n, 1), jnp.float32),
            jax.ShapeDtypeStruct((n, 1), jnp.float32),
            jax.ShapeDtypeStruct((n_tiles, 1, 16), jnp.int32),
            jax.ShapeDtypeStruct((n_tiles, 1, 16), jnp.int32),
        ],
        compiler_params=pltpu.CompilerParams(
            dimension_semantics=("arbitrary",),
        ),
        interpret=_INTERPRET,
    )(x_flat, gate_w)


def _shared_body(x_ref, w1_ref, w2_ref, w3_ref, sh_ref):
    x = x_ref[...]
    gate = _silu(jnp.dot(x, w1_ref[...], preferred_element_type=jnp.float32))
    up = jnp.dot(x, w2_ref[...], preferred_element_type=jnp.float32)
    sh_ref[...] = jnp.dot(gate * up, w3_ref[...], preferred_element_type=jnp.float32)


def _shared(x_part, sw1, sw2, sw3):
    n = x_part.shape[0]
    n_tiles = n // ROW_TILE
    return pl.pallas_call(
        _shared_body,
        grid=(n_tiles,),
        in_specs=[
            pl.BlockSpec((ROW_TILE, D_MODEL), lambda t: (t, 0)),
            pl.BlockSpec((D_MODEL, D_FF), lambda t: (0, 0)),
            pl.BlockSpec((D_MODEL, D_FF), lambda t: (0, 0)),
            pl.BlockSpec((D_FF, D_MODEL), lambda t: (0, 0)),
        ],
        out_specs=pl.BlockSpec((ROW_TILE, D_MODEL), lambda t: (t, 0)),
        out_shape=jax.ShapeDtypeStruct((n, D_MODEL), jnp.float32),
        compiler_params=pltpu.CompilerParams(
            dimension_semantics=("arbitrary",),
        ),
        interpret=_INTERPRET,
    )(x_part, sw1, sw2, sw3)


def _wid():
    return lax.axis_index("s") * NC + lax.axis_index("c")


def _splat(v, k):
    return v[jnp.full((16,), k, jnp.int32)]


def _cumsum16(v, iota):
    for k in (1, 2, 4, 8):
        sv = v[jnp.maximum(iota - k, 0)]
        v = v + jnp.where(iota >= k, sv, 0)
    return v


def _make_sort_dispatch_kernel(n_tok, n_assign, te_len, pad_total):
    chunk = n_assign // NW
    rows = 32
    n_rows_chunks = chunk // rows
    mesh = plsc.VectorSubcoreMesh(core_axis_name="c", subcore_axis_name="s")

    @functools.partial(
        pl.kernel,
        out_type=[
            jax.ShapeDtypeStruct((n_assign // rows, rows), jnp.int32),
            jax.ShapeDtypeStruct((te_len,), jnp.int32),
            jax.ShapeDtypeStruct((te_len,), jnp.int32),
            jax.ShapeDtypeStruct((pad_total, D_MODEL), jnp.float32),
        ],
        mesh=mesh,
        scratch_types=[
            pltpu.VMEM((chunk,), jnp.int32),
            pltpu.VMEM((NW, 16), jnp.int32),
            pltpu.VMEM((n_rows_chunks, rows), jnp.int32),
            pltpu.VMEM((te_len,), jnp.int32),
            pltpu.VMEM((te_len,), jnp.int32),
            pltpu.VMEM((rows, D_MODEL), jnp.float32),
            pltpu.VMEM((rows, D_MODEL), jnp.float32),
            pltpu.SemaphoreType.DMA,
            pltpu.SemaphoreType.DMA,
        ],
    )
    def sort_dispatch_kernel(e0_hbm, e1_hbm, h0_hbm, h1_hbm, x_hbm, dst_hbm,
                             te_hbm, af_hbm, disp_hbm, ids_v, allh_v, dst_v,
                             te_v, af_v, rows_v0, rows_v1, sem0, sem1):
        w = _wid()

        @pl.when(w < NS)
        def _():
            pltpu.sync_copy(e0_hbm.at[pl.ds((w % NS) * chunk, chunk)], ids_v)

        @pl.when(w >= NS)
        def _():
            pltpu.sync_copy(e1_hbm.at[pl.ds((w % NS) * chunk, chunk)], ids_v)

        pltpu.sync_copy(h0_hbm, allh_v.at[pl.ds(0, NS)])
        pltpu.sync_copy(h1_hbm, allh_v.at[pl.ds(NS, NS)])
        iota = lax.iota(jnp.int32, 16)
        tot = jnp.zeros((16,), jnp.int32)
        pre = jnp.zeros((16,), jnp.int32)
        for ww in range(NW):
            row = allh_v[ww]
            tot = tot + row
            pre = pre + row * jnp.where(ww < w, 1, 0)
        padded = (tot + (D_TILE - 1)) & jnp.int32(-D_TILE)
        csum = _cumsum16(padded, iota)
        off = csum - padded
        start = off + pre

        cnt = jnp.zeros((16,), jnp.int32)
        for j in range(chunk // 16):
            v = ids_v[pl.ds(j * 16, 16)]
            rank = jnp.zeros((16,), jnp.int32)
            for k in range(1, 16):
                sv = v[jnp.maximum(iota - k, 0)]
                rank = rank + jnp.where((iota >= k) & (sv == v), 1, 0)
            dst_v[j // 2, pl.ds((j % 2) * 16, 16)] = start[v] + cnt[v] + rank
            cc = jnp.zeros((16,), jnp.int32)
            for k in range(16):
                cc = cc + jnp.where(iota == _splat(v, k), 1, 0)
            cnt = cnt + cc
        pltpu.sync_copy(dst_v, dst_hbm.at[pl.ds(w * n_rows_chunks, n_rows_chunks)])

        tok_per_slotblock = n_tok // NS
        bufs = (rows_v0, rows_v1)
        sems = (sem0, sem1)
        cps = [None, None]
        for r in range(n_rows_chunks):
            b = r % 2
            if cps[b] is not None:
                cps[b].wait()
            tb = (w % NS) * tok_per_slotblock + r * rows
            pltpu.sync_copy(x_hbm.at[pl.ds(tb, rows)], bufs[b])
            cps[b] = pltpu.async_copy(bufs[b], disp_hbm.at[dst_v.at[r]], sems[b])
        for cp in cps:
            if cp is not None:
                cp.wait()

        @pl.when(w == 0)
        def _():
            used_end = off + tot
            for g in range(te_len // 16):
                tstart = (iota + g * 16) * D_TILE
                acc = jnp.zeros((16,), jnp.int32)
                for e in range(NUM_E):
                    acc = acc + jnp.where(_splat(off, e) <= tstart, 1, 0)
                ex = acc - 1
                te_v[pl.ds(g * 16, 16)] = ex
                af_v[pl.ds(g * 16, 16)] = jnp.where(tstart < used_end[ex], 1, 0)
            pltpu.sync_copy(te_v, te_hbm)
            pltpu.sync_copy(af_v, af_hbm)

    return sort_dispatch_kernel


def _expert_ffn_body(te_ref, af_ref, x_ref, w1_ref, w2_ref, w3_ref, out_ref):
    del te_ref
    t = pl.program_id(0)

    @pl.when(af_ref[t] == 1)
    def _():
        x = x_ref[...]
        gate = _silu(jnp.dot(x, w1_ref[0], preferred_element_type=jnp.float32))
        up = jnp.dot(x, w2_ref[0], preferred_element_type=jnp.float32)
        out_ref[...] = jnp.dot(gate * up, w3_ref[0], preferred_element_type=jnp.float32)


def _expert_ffn(te, af, disp, ew1, ew2, ew3, n_tiles):
    pad_total = disp.shape[0]
    grid_spec = pltpu.PrefetchScalarGridSpec(
        num_scalar_prefetch=2,
        grid=(n_tiles,),
        in_specs=[
            pl.BlockSpec((D_TILE, D_MODEL), lambda t, te, af: (t, 0)),
            pl.BlockSpec((1, D_MODEL, D_FF), lambda t, te, af: (te[t], 0, 0)),
            pl.BlockSpec((1, D_MODEL, D_FF), lambda t, te, af: (te[t], 0, 0)),
            pl.BlockSpec((1, D_FF, D_MODEL), lambda t, te, af: (te[t], 0, 0)),
        ],
        out_specs=pl.BlockSpec((D_TILE, D_MODEL), lambda t, te, af: (t, 0)),
    )
    return pl.pallas_call(
        _expert_ffn_body,
        grid_spec=grid_spec,
        out_shape=jax.ShapeDtypeStruct((pad_total, D_MODEL), jnp.float32),
        compiler_params=pltpu.CompilerParams(
            dimension_semantics=("arbitrary",),
            vmem_limit_bytes=100 * 1024 * 1024,
        ),
        interpret=_INTERPRET,
    )(te, af, disp, ew1, ew2, ew3)


def _make_gather_kernel(n_assign, pad_total):
    rows = 32
    n_rows_chunks = n_assign // rows // NW
    mesh = plsc.VectorSubcoreMesh(core_axis_name="c", subcore_axis_name="s")

    @functools.partial(
        pl.kernel,
        out_type=jax.ShapeDtypeStruct((n_assign, D_MODEL), jnp.float32),
        mesh=mesh,
        scratch_types=[
            pltpu.VMEM((rows, D_MODEL), jnp.float32),
            pltpu.VMEM((rows, D_MODEL), jnp.float32),
            pltpu.VMEM((n_rows_chunks, rows), jnp.int32),
            pltpu.SemaphoreType.DMA,
            pltpu.SemaphoreType.DMA,
        ],
    )
    def gather_kernel(eout_hbm, dst_hbm, g_hbm, rows_v0, rows_v1, idx_v,
                      sem0, sem1):
        w = _wid()
        pltpu.sync_copy(
            dst_hbm.at[pl.ds(w * n_rows_chunks, n_rows_chunks)], idx_v)
        bufs = (rows_v0, rows_v1)
        sems = (sem0, sem1)
        cps = [None, None]
        for c in range(n_rows_chunks):
            b = c % 2
            cps[b] = pltpu.async_copy(eout_hbm.at[idx_v.at[c]], bufs[b], sems[b])
            if c > 0:
                cps[1 - b].wait()
                rr = w * n_rows_chunks + c - 1
                pltpu.sync_copy(bufs[1 - b], g_hbm.at[pl.ds(rr * rows, rows)])
        last = n_rows_chunks - 1
        cps[last % 2].wait()
        pltpu.sync_copy(bufs[last % 2],
                        g_hbm.at[pl.ds((w * n_rows_chunks + last) * rows, rows)])

    return gather_kernel


def _combine_body(sh_ref, g0_ref, g1_ref, w0_ref, w1_ref, out_ref):
    out_ref[...] = (sh_ref[...]
                    + w0_ref[...] * g0_ref[...]
                    + w1_ref[...] * g1_ref[...])


def _combine(shared_out, g, w0c, w1c, n_tok):
    n_tiles = n_tok // ROW_TILE
    return pl.pallas_call(
        _combine_body,
        grid=(n_tiles,),
        in_specs=[
            pl.BlockSpec((ROW_TILE, D_MODEL), lambda t: (t, 0)),
            pl.BlockSpec((ROW_TILE, D_MODEL), lambda t: (t, 0)),
            pl.BlockSpec((ROW_TILE, D_MODEL), lambda t: (t + n_tok // ROW_TILE, 0)),
            pl.BlockSpec((ROW_TILE, 1), lambda t: (t, 0)),
            pl.BlockSpec((ROW_TILE, 1), lambda t: (t, 0)),
        ],
        out_specs=pl.BlockSpec((ROW_TILE, D_MODEL), lambda t: (t, 0)),
        out_shape=jax.ShapeDtypeStruct((n_tok, D_MODEL), jnp.float32),
        compiler_params=pltpu.CompilerParams(
            dimension_semantics=("arbitrary",),
        ),
        interpret=_INTERPRET,
    )(shared_out, g, g, w0c, w1c)


def kernel(x, shared_w1, shared_w2, shared_w3, expert_w1, expert_w2, expert_w3, gate_w):
    Bn, Tn, C = x.shape
    n_tok = Bn * Tn
    n_assign = 2 * n_tok
    pad_total = n_assign + NUM_E * D_TILE
    n_tiles = n_assign // D_TILE + NUM_E
    te_len = 64

    x_flat = x.reshape(n_tok, C)

    a0, a1, w0c, w1c, h0, h1 = _router(x_flat, gate_w)

    dst2d, te, af, disp = _make_sort_dispatch_kernel(
        n_tok, n_assign, te_len, pad_total)(
            a0.reshape(-1), a1.reshape(-1),
            h0.reshape(NS, 16), h1.reshape(NS, 16), x_flat)
    shared_out = _shared(x_flat, shared_w1, shared_w2, shared_w3)
    eout = _expert_ffn(te, af, disp, expert_w1, expert_w2, expert_w3, n_tiles)
    g = _make_gather_kernel(n_assign, pad_total)(eout, dst2d)
    out = _combine(shared_out, g, w0c, w1c, n_tok)

    final_out = out.reshape(Bn, Tn, C)
    aux_loss = jnp.array(0.0, dtype=jnp.float32)
    return (final_out, aux_loss)


# speedup vs baseline: 1.1398x; 1.0235x over previous
import functools

import jax
import jax.numpy as jnp
from jax import lax
from jax.experimental import pallas as pl
from jax.experimental.pallas import tpu as pltpu
from jax.experimental.pallas import tpu_sc as plsc

D_MODEL = 1024
D_FF = 2048
NUM_E = 8
ROW_TILE = 256
D_TILE = 512

NC = 2
NS = 16
NW = NC * NS

_INTERPRET = False


def _silu(v):
    return v * (1.0 / (1.0 + jnp.exp(-v)))


R_TILE = 1024
SC_CHUNK = 256


def _router_body(x_ref, gate_ref, a0_ref, a1_ref, w0_ref, w1c_ref,
                 h0_ref, h1_ref):
    x = x_ref[...]
    rows = x.shape[0]
    nsub = rows // SC_CHUNK
    logits = jnp.dot(x, gate_ref[...], preferred_element_type=jnp.float32)
    iota = lax.broadcasted_iota(jnp.int32, (rows, NUM_E), 1)
    m0 = jnp.max(logits, axis=1, keepdims=True)
    a0 = jnp.min(jnp.where(logits == m0, iota, NUM_E), axis=1, keepdims=True)
    masked = jnp.where(iota == a0, jnp.float32(-1e30), logits)
    m1 = jnp.max(masked, axis=1, keepdims=True)
    a1 = jnp.min(jnp.where(masked == m1, iota, NUM_E), axis=1, keepdims=True)
    t = jnp.exp(m1 - m0)
    a0_ref[...] = a0
    a1_ref[...] = a1
    w0_ref[...] = 1.0 / (1.0 + t)
    w1c_ref[...] = t / (1.0 + t)
    iota16 = lax.broadcasted_iota(jnp.int32, (rows, 16), 1)
    h0 = jnp.where(iota16 == a0, 1, 0).reshape(nsub, SC_CHUNK, 16)
    h1 = jnp.where(iota16 == a1, 1, 0).reshape(nsub, SC_CHUNK, 16)
    h0_ref[...] = jnp.sum(h0, axis=1, keepdims=True)
    h1_ref[...] = jnp.sum(h1, axis=1, keepdims=True)


def _router(x_flat, gate_w):
    n = x_flat.shape[0]
    n_tiles = n // R_TILE
    nsub = R_TILE // SC_CHUNK
    return pl.pallas_call(
        _router_body,
        grid=(n_tiles,),
        in_specs=[
            pl.BlockSpec((R_TILE, D_MODEL), lambda t: (t, 0)),
            pl.BlockSpec((D_MODEL, NUM_E), lambda t: (0, 0)),
        ],
        out_specs=[
            pl.BlockSpec((R_TILE, 1), lambda t: (t, 0)),
            pl.BlockSpec((R_TILE, 1), lambda t: (t, 0)),
            pl.BlockSpec((R_TILE, 1), lambda t: (t, 0)),
            pl.BlockSpec((R_TILE, 1), lambda t: (t, 0)),
            pl.BlockSpec((nsub, 1, 16), lambda t: (t, 0, 0)),
            pl.BlockSpec((nsub, 1, 16), lambda t: (t, 0, 0)),
        ],
        out_shape=[
            jax.ShapeDtypeStruct((n, 1), jnp.int32),
            jax.ShapeDtypeStruct((n, 1), jnp.int32),
            jax.ShapeDtypeStruct((n, 1), jnp.float32),
            jax.ShapeDtypeStruct((n, 1), jnp.float32),
            jax.ShapeDtypeStruct((n // SC_CHUNK, 1, 16), jnp.int32),
            jax.ShapeDtypeStruct((n // SC_CHUNK, 1, 16), jnp.int32),
        ],
        compiler_params=pltpu.CompilerParams(
            dimension_semantics=("arbitrary",),
        ),
        interpret=_INTERPRET,
    )(x_flat, gate_w)


def _shared_body(x_ref, w1_ref, w2_ref, w3_ref, sh_ref):
    x = x_ref[...]
    gate = _silu(jnp.dot(x, w1_ref[...], preferred_element_type=jnp.float32))
    up = jnp.dot(x, w2_ref[...], preferred_element_type=jnp.float32)
    sh_ref[...] = jnp.dot(gate * up, w3_ref[...], preferred_element_type=jnp.float32)


def _shared(x_part, sw1, sw2, sw3):
    n = x_part.shape[0]
    n_tiles = n // ROW_TILE
    return pl.pallas_call(
        _shared_body,
        grid=(n_tiles,),
        in_specs=[
            pl.BlockSpec((ROW_TILE, D_MODEL), lambda t: (t, 0)),
            pl.BlockSpec((D_MODEL, D_FF), lambda t: (0, 0)),
            pl.BlockSpec((D_MODEL, D_FF), lambda t: (0, 0)),
            pl.BlockSpec((D_FF, D_MODEL), lambda t: (0, 0)),
        ],
        out_specs=pl.BlockSpec((ROW_TILE, D_MODEL), lambda t: (t, 0)),
        out_shape=jax.ShapeDtypeStruct((n, D_MODEL), jnp.float32),
        compiler_params=pltpu.CompilerParams(
            dimension_semantics=("arbitrary",),
        ),
        interpret=_INTERPRET,
    )(x_part, sw1, sw2, sw3)


def _wid():
    return lax.axis_index("s") * NC + lax.axis_index("c")


def _splat(v, k):
    return v[jnp.full((16,), k, jnp.int32)]


def _cumsum16(v, iota):
    for k in (1, 2, 4, 8):
        sv = v[jnp.maximum(iota - k, 0)]
        v = v + jnp.where(iota >= k, sv, 0)
    return v


def _make_sort_dispatch_kernel(n_tok, n_assign, te_len, pad_total):
    chunk = n_assign // NW
    rows = 32
    n_rows_chunks = chunk // rows
    mesh = plsc.VectorSubcoreMesh(core_axis_name="c", subcore_axis_name="s")

    @functools.partial(
        pl.kernel,
        out_type=[
            jax.ShapeDtypeStruct((n_assign // rows, rows), jnp.int32),
            jax.ShapeDtypeStruct((te_len,), jnp.int32),
            jax.ShapeDtypeStruct((te_len,), jnp.int32),
            jax.ShapeDtypeStruct((pad_total, D_MODEL), jnp.float32),
        ],
        mesh=mesh,
        scratch_types=[
            pltpu.VMEM((chunk,), jnp.int32),
            pltpu.VMEM((NW, 16), jnp.int32),
            pltpu.VMEM((n_rows_chunks, rows), jnp.int32),
            pltpu.VMEM((te_len,), jnp.int32),
            pltpu.VMEM((te_len,), jnp.int32),
            pltpu.VMEM((rows, D_MODEL), jnp.float32),
            pltpu.VMEM((rows, D_MODEL), jnp.float32),
            pltpu.SemaphoreType.DMA,
            pltpu.SemaphoreType.DMA,
        ],
    )
    def sort_dispatch_kernel(e0_hbm, e1_hbm, h0_hbm, h1_hbm, x_hbm, dst_hbm,
                             te_hbm, af_hbm, disp_hbm, ids_v, allh_v, dst_v,
                             te_v, af_v, rows_v0, rows_v1, sem0, sem1):
        w = _wid()

        @pl.when(w < NS)
        def _():
            pltpu.sync_copy(e0_hbm.at[pl.ds((w % NS) * chunk, chunk)], ids_v)

        @pl.when(w >= NS)
        def _():
            pltpu.sync_copy(e1_hbm.at[pl.ds((w % NS) * chunk, chunk)], ids_v)

        pltpu.sync_copy(h0_hbm, allh_v.at[pl.ds(0, NS)])
        pltpu.sync_copy(h1_hbm, allh_v.at[pl.ds(NS, NS)])
        iota = lax.iota(jnp.int32, 16)
        tot = jnp.zeros((16,), jnp.int32)
        pre = jnp.zeros((16,), jnp.int32)
        for ww in range(NW):
            row = allh_v[ww]
            tot = tot + row
            pre = pre + row * jnp.where(ww < w, 1, 0)
        padded = (tot + (D_TILE - 1)) & jnp.int32(-D_TILE)
        csum = _cumsum16(padded, iota)
        off = csum - padded
        start = off + pre

        cnt = jnp.zeros((16,), jnp.int32)
        for j in range(chunk // 16):
            v = ids_v[pl.ds(j * 16, 16)]
            rank = jnp.zeros((16,), jnp.int32)
            for k in range(1, 16):
                sv = v[jnp.maximum(iota - k, 0)]
                rank = rank + jnp.where((iota >= k) & (sv == v), 1, 0)
            dst_v[j // 2, pl.ds((j % 2) * 16, 16)] = start[v] + cnt[v] + rank
            cc = jnp.zeros((16,), jnp.int32)
            for k in range(16):
                cc = cc + jnp.where(iota == _splat(v, k), 1, 0)
            cnt = cnt + cc
        pltpu.sync_copy(dst_v, dst_hbm.at[pl.ds(w * n_rows_chunks, n_rows_chunks)])

        tok_per_slotblock = n_tok // NS
        bufs = (rows_v0, rows_v1)
        sems = (sem0, sem1)
        cps = [None, None]
        for r in range(n_rows_chunks):
            b = r % 2
            if cps[b] is not None:
                cps[b].wait()
            tb = (w % NS) * tok_per_slotblock + r * rows
            pltpu.sync_copy(x_hbm.at[pl.ds(tb, rows)], bufs[b])
            cps[b] = pltpu.async_copy(bufs[b], disp_hbm.at[dst_v.at[r]], sems[b])
        for cp in cps:
            if cp is not None:
                cp.wait()

        @pl.when(w == 0)
        def _():
            used_end = off + tot
            for g in range(te_len // 16):
                tstart = (iota + g * 16) * D_TILE
                acc = jnp.zeros((16,), jnp.int32)
                for e in range(NUM_E):
                    acc = acc + jnp.where(_splat(off, e) <= tstart, 1, 0)
                ex = acc - 1
                te_v[pl.ds(g * 16, 16)] = ex
                af_v[pl.ds(g * 16, 16)] = jnp.where(tstart < used_end[ex], 1, 0)
            pltpu.sync_copy(te_v, te_hbm)
            pltpu.sync_copy(af_v, af_hbm)

    return sort_dispatch_kernel


def _expert_ffn_body(te_ref, af_ref, x_ref, w1_ref, w2_ref, w3_ref, out_ref):
    del te_ref
    t = pl.program_id(0)

    @pl.when(af_ref[t] == 1)
    def _():
        x = x_ref[...]
        gate = _silu(jnp.dot(x, w1_ref[0], preferred_element_type=jnp.float32))
        up = jnp.dot(x, w2_ref[0], preferred_element_type=jnp.float32)
        out_ref[...] = jnp.dot(gate * up, w3_ref[0], preferred_element_type=jnp.float32)


def _expert_ffn(te, af, disp, ew1, ew2, ew3, n_tiles):
    pad_total = disp.shape[0]
    grid_spec = pltpu.PrefetchScalarGridSpec(
        num_scalar_prefetch=2,
        grid=(n_tiles,),
        in_specs=[
            pl.BlockSpec((D_TILE, D_MODEL), lambda t, te, af: (t, 0)),
            pl.BlockSpec((1, D_MODEL, D_FF), lambda t, te, af: (te[t], 0, 0)),
            pl.BlockSpec((1, D_MODEL, D_FF), lambda t, te, af: (te[t], 0, 0)),
            pl.BlockSpec((1, D_FF, D_MODEL), lambda t, te, af: (te[t], 0, 0)),
        ],
        out_specs=pl.BlockSpec((D_TILE, D_MODEL), lambda t, te, af: (t, 0)),
    )
    return pl.pallas_call(
        _expert_ffn_body,
        grid_spec=grid_spec,
        out_shape=jax.ShapeDtypeStruct((pad_total, D_MODEL), jnp.float32),
        compiler_params=pltpu.CompilerParams(
            dimension_semantics=("arbitrary",),
            vmem_limit_bytes=100 * 1024 * 1024,
        ),
        interpret=_INTERPRET,
    )(te, af, disp, ew1, ew2, ew3)


def _make_gather_kernel(n_assign, pad_total):
    rows = 32
    n_rows_chunks = n_assign // rows // NW
    mesh = plsc.VectorSubcoreMesh(core_axis_name="c", subcore_axis_name="s")

    @functools.partial(
        pl.kernel,
        out_type=jax.ShapeDtypeStruct((n_assign, D_MODEL), jnp.float32),
        mesh=mesh,
        scratch_types=[
            pltpu.VMEM((rows, D_MODEL), jnp.float32),
            pltpu.VMEM((rows, D_MODEL), jnp.float32),
            pltpu.VMEM((n_rows_chunks, rows), jnp.int32),
            pltpu.SemaphoreType.DMA,
            pltpu.SemaphoreType.DMA,
        ],
    )
    def gather_kernel(eout_hbm, dst_hbm, g_hbm, rows_v0, rows_v1, idx_v,
                      sem0, sem1):
        w = _wid()
        pltpu.sync_copy(
            dst_hbm.at[pl.ds(w * n_rows_chunks, n_rows_chunks)], idx_v)
        bufs = (rows_v0, rows_v1)
        sems = (sem0, sem1)
        cps = [None, None]
        for c in range(n_rows_chunks):
            b = c % 2
            cps[b] = pltpu.async_copy(eout_hbm.at[idx_v.at[c]], bufs[b], sems[b])
            if c > 0:
                cps[1 - b].wait()
                rr = w * n_rows_chunks + c - 1
                pltpu.sync_copy(bufs[1 - b], g_hbm.at[pl.ds(rr * rows, rows)])
        last = n_rows_chunks - 1
        cps[last % 2].wait()
        pltpu.sync_copy(bufs[last % 2],
                        g_hbm.at[pl.ds((w * n_rows_chunks + last) * rows, rows)])

    return gather_kernel


def _combine_body(sh_ref, g0_ref, g1_ref, w0_ref, w1_ref, out_ref):
    out_ref[...] = (sh_ref[...]
                    + w0_ref[...] * g0_ref[...]
                    + w1_ref[...] * g1_ref[...])


def _combine(shared_out, g, w0c, w1c, n_tok):
    n_tiles = n_tok // ROW_TILE
    return pl.pallas_call(
        _combine_body,
        grid=(n_tiles,),
        in_specs=[
            pl.BlockSpec((ROW_TILE, D_MODEL), lambda t: (t, 0)),
            pl.BlockSpec((ROW_TILE, D_MODEL), lambda t: (t, 0)),
            pl.BlockSpec((ROW_TILE, D_MODEL), lambda t: (t + n_tok // ROW_TILE, 0)),
            pl.BlockSpec((ROW_TILE, 1), lambda t: (t, 0)),
            pl.BlockSpec((ROW_TILE, 1), lambda t: (t, 0)),
        ],
        out_specs=pl.BlockSpec((ROW_TILE, D_MODEL), lambda t: (t, 0)),
        out_shape=jax.ShapeDtypeStruct((n_tok, D_MODEL), jnp.float32),
        compiler_params=pltpu.CompilerParams(
            dimension_semantics=("arbitrary",),
        ),
        interpret=_INTERPRET,
    )(shared_out, g, g, w0c, w1c)


def kernel(x, shared_w1, shared_w2, shared_w3, expert_w1, expert_w2, expert_w3, gate_w):
    Bn, Tn, C = x.shape
    n_tok = Bn * Tn
    n_assign = 2 * n_tok
    pad_total = n_assign + NUM_E * D_TILE
    n_tiles = n_assign // D_TILE + NUM_E
    te_len = 64

    x_flat = x.reshape(n_tok, C)

    a0, a1, w0c, w1c, h0, h1 = _router(x_flat, gate_w)

    dst2d, te, af, disp = _make_sort_dispatch_kernel(
        n_tok, n_assign, te_len, pad_total)(
            a0.reshape(-1), a1.reshape(-1),
            h0.reshape(NS, 16), h1.reshape(NS, 16), x_flat)
    shared_out = _shared(x_flat, shared_w1, shared_w2, shared_w3)
    eout = _expert_ffn(te, af, disp, expert_w1, expert_w2, expert_w3, n_tiles)
    g = _make_gather_kernel(n_assign, pad_total)(eout, dst2d)
    out = _combine(shared_out, g, w0c, w1c, n_tok)

    final_out = out.reshape(Bn, Tn, C)
    aux_loss = jnp.array(0.0, dtype=jnp.float32)
    return (final_out, aux_loss)
